# Initial kernel scaffold; baseline (speedup 1.0000x reference)
import jax
import jax.numpy as jnp
from jax.experimental import pallas as pl


def kernel(input_features, edge_index, edge_weight, W, b):
    raise NotImplementedError("write your pallas kernel here")


# speedup vs baseline: 4.4221x; 4.4221x over previous
import functools

import jax
import jax.numpy as jnp
from jax import lax
from jax.experimental import pallas as pl
from jax.experimental.pallas import tpu as pltpu
from jax.experimental.pallas import tpu_sc as plsc

N = 10000
E = 320000
F = 128

NC = 2
NS = 16
L = 16

NW = NC * NS
E_PER_W = E // NW
CHUNK = 80
N_CHUNKS = E_PER_W // CHUNK
N_PAD = 10240
ROWS_PER_TILE = N_PAD // NS

MM_BLOCK = 1000

_BCAST_DNUMS = lax.GatherDimensionNumbers(
    offset_dims=(), collapsed_slice_dims=(0,), start_index_map=(0,))


def _bcast_lane(v16, j):
    idx = jnp.full((L, 1), j, jnp.int32)
    return lax.gather(v16, idx, _BCAST_DNUMS, (1,),
                      mode=lax.GatherScatterMode.PROMISE_IN_BOUNDS)


def _mm_body(x_ref, w_ref, o_ref):
    o_ref[:] = jnp.dot(x_ref[:], w_ref[:], preferred_element_type=jnp.float32)


def _matmul(x, W):
    return pl.pallas_call(
        _mm_body,
        grid=(N // MM_BLOCK,),
        in_specs=[
            pl.BlockSpec((MM_BLOCK, F), lambda i: (i, 0)),
            pl.BlockSpec((F, F), lambda i: (0, 0)),
        ],
        out_specs=pl.BlockSpec((MM_BLOCK, F), lambda i: (i, 0)),
        out_shape=jax.ShapeDtypeStruct((N, F), jnp.float32),
    )(x, W)


def _agg_body(support, srcs, dsts, ws, zeros, out,
              acc, src_v, dst_v, w_v, rows_v, gsem):
    cid = lax.axis_index("c")
    sid = lax.axis_index("s")
    wid = sid * NC + cid

    pltpu.sync_copy(zeros, acc.at[pl.ds(sid * ROWS_PER_TILE, ROWS_PER_TILE)])
    plsc.subcore_barrier()

    ebase = pl.multiple_of(wid * E_PER_W, 8)

    def chunk_body(c, carry):
        base = pl.multiple_of(ebase + c * CHUNK, 8)
        pltpu.sync_copy(srcs.at[pl.ds(base, CHUNK)], src_v)
        pltpu.sync_copy(dsts.at[pl.ds(base, CHUNK)], dst_v)
        pltpu.sync_copy(ws.at[pl.ds(base, CHUNK)], w_v)
        pltpu.async_copy(support.at[src_v], rows_v, gsem).wait()
        for g in range(CHUNK // L):
            w16 = w_v[pl.ds(g * L, L)]
            for j in range(L):
                wb = _bcast_lane(w16, j)
                e = g * L + j
                for k in range(F // L):
                    rows_v[e, pl.ds(k * L, L)] = rows_v[e, pl.ds(k * L, L)] * wb
        pltpu.sync_copy(rows_v, acc.at[dst_v], add=True)
        return carry

    lax.fori_loop(0, N_CHUNKS, chunk_body, 0)

    plsc.subcore_barrier()
    pltpu.sync_copy(acc.at[pl.ds(sid * ROWS_PER_TILE, ROWS_PER_TILE)],
                    out.at[cid, pl.ds(sid * ROWS_PER_TILE, ROWS_PER_TILE)])


_agg = pl.kernel(
    _agg_body,
    out_type=jax.ShapeDtypeStruct((NC, N_PAD, F), jnp.float32),
    mesh=plsc.VectorSubcoreMesh(core_axis_name="c", subcore_axis_name="s"),
    scratch_types=[
        pltpu.VMEM_SHARED((N_PAD, F), jnp.float32),
        pltpu.VMEM((CHUNK,), jnp.int32),
        pltpu.VMEM((CHUNK,), jnp.int32),
        pltpu.VMEM((CHUNK,), jnp.float32),
        pltpu.VMEM((CHUNK, F), jnp.float32),
        pltpu.SemaphoreType.DMA,
    ],
)


def _comb_body(p_ref, b_ref, o_ref):
    o_ref[:] = p_ref[0] + p_ref[1] + b_ref[:]


def _combine(partials, b2d):
    return pl.pallas_call(
        _comb_body,
        grid=(N // MM_BLOCK,),
        in_specs=[
            pl.BlockSpec((NC, MM_BLOCK, F), lambda i: (0, i, 0)),
            pl.BlockSpec((1, F), lambda i: (0, 0)),
        ],
        out_specs=pl.BlockSpec((MM_BLOCK, F), lambda i: (i, 0)),
        out_shape=jax.ShapeDtypeStruct((N, F), jnp.float32),
    )(partials, b2d)


def kernel(input_features, edge_index, edge_weight, W, b):
    dst = edge_index[0].astype(jnp.int32)
    src = edge_index[1].astype(jnp.int32)
    support = _matmul(input_features, W)
    zeros = jnp.zeros((ROWS_PER_TILE, F), jnp.float32)
    partials = _agg(support, src, dst, edge_weight, zeros)
    return _combine(partials, b.reshape(1, F))
